# baseline (speedup 1.0000x reference)
import jax
import jax.numpy as jnp
from jax import lax
from jax.experimental import pallas as pl
from jax.experimental.pallas import tpu as pltpu
from jax.experimental.pallas import tpu_sc as plsc

R = 128
N = 32768
K = 64
NC = 2
NS = 16
NW = NC * NS
ROWS_PER_W = R // NW
L = 16
NGRP = 16
STRIDE = NGRP * L
SEGLEN = N // STRIDE
NJ = SEGLEN // L
P1_UNROLL = 4

NEG_INF = float("-inf")


def _treemax(vs):
    while len(vs) > 1:
        vs = [jnp.maximum(vs[i], vs[i + 1]) for i in range(0, len(vs) - 1, 2)] + (
            [vs[-1]] if len(vs) % 2 else [])
    return vs[0]


def _tec_body(x_hbm, out_hbm, buf0, buf1, outbuf, smax, sem):
    wid = lax.axis_index("s") * NC + lax.axis_index("c")
    iota = lax.iota(jnp.int32, L)
    neg_vec = jnp.full((L,), NEG_INF, jnp.float32)
    lane0 = iota == 0
    row0 = wid * ROWS_PER_W

    def row_slice(r):
        return x_hbm.at[pl.ds(pl.multiple_of((row0 + r) * N, N), N)]

    bufs = [buf0, buf1]
    pltpu.sync_copy(row_slice(0), buf0)

    for r in range(ROWS_PER_W):
        rowbuf = bufs[r % 2]
        cp = None
        if r + 1 < ROWS_PER_W:
            cp = pltpu.make_async_copy(row_slice(r + 1), bufs[(r + 1) % 2],
                                       sem)
            cp.start()

        def p1_body(j, ms):
            ms = list(ms)
            for u in range(P1_UNROLL):
                base = pl.multiple_of((j * P1_UNROLL + u) * STRIDE, STRIDE)
                for g in range(NGRP):
                    ms[g] = jnp.maximum(ms[g], rowbuf[pl.ds(base + g * L, L)])
            return tuple(ms)

        init = tuple(jnp.full((L,), NEG_INF, jnp.float32) for _ in range(NGRP))
        segmax = lax.fori_loop(0, SEGLEN // P1_UNROLL, p1_body, init)

        t = jnp.full((L,), NEG_INF, jnp.float32)
        for g in range(NGRP):
            smax[pl.ds(g * L, L)] = segmax[g]
            t = jnp.where(iota == g, jnp.max(segmax[g]), t)

        def ext_body(i, carry):
            t, o0, o1, o2, o3 = carry
            gm = jnp.max(t)
            g_spl = plsc.all_reduce_ffs(t >= gm) + jnp.zeros((L,), jnp.int32)
            gvec = plsc.load_gather(smax, [g_spl * L + iota])
            l_spl = plsc.all_reduce_ffs(gvec >= gm) + jnp.zeros((L,), jnp.int32)
            base = g_spl * L + l_spl

            done = jnp.zeros((L,), jnp.bool_)
            kidx = jnp.zeros((L,), jnp.int32)
            kmask = jnp.zeros((L,), jnp.bool_)
            nv = []
            for ja in range(NJ):
                idx = (ja * L + iota) * STRIDE + base
                v = plsc.load_gather(rowbuf, [idx])
                eq = v >= gm
                hit = plsc.all_reduce_population_count(eq) > 0
                take = hit & (~done)
                f = plsc.all_reduce_ffs(eq)
                km = (iota == f) & take
                done = done | hit
                kidx = jnp.where(km, idx, kidx)
                kmask = kmask | km
                nv.append(jnp.where(km, neg_vec, v))
            plsc.store_scatter(rowbuf, [kidx], neg_vec, mask=kmask)
            newmax_s = jnp.max(_treemax(nv))

            gvec2 = jnp.where(iota == l_spl, newmax_s, gvec)
            plsc.store_scatter(smax, [g_spl * L + iota], gvec2)
            t = jnp.where(iota == g_spl, jnp.max(gvec2), t)

            o0 = jnp.where((i < 16) & (iota == i), gm, o0)
            o1 = jnp.where((i >= 16) & (i < 32) & (iota == i - 16), gm, o1)
            o2 = jnp.where((i >= 32) & (i < 48) & (iota == i - 32), gm, o2)
            o3 = jnp.where((i >= 48) & (iota == i - 48), gm, o3)
            return t, o0, o1, o2, o3

        z = jnp.full((L,), NEG_INF, jnp.float32)
        t, o0, o1, o2, o3 = lax.fori_loop(0, K, ext_body, (t, z, z, z, z))
        outbuf[pl.ds(r * K, L)] = o0
        outbuf[pl.ds(r * K + 16, L)] = o1
        outbuf[pl.ds(r * K + 32, L)] = o2
        outbuf[pl.ds(r * K + 48, L)] = o3

        if cp is not None:
            cp.wait()

    pltpu.sync_copy(
        outbuf, out_hbm.at[pl.ds(pl.multiple_of(wid * ROWS_PER_W * K, K),
                                 ROWS_PER_W * K)])


def kernel(x):
    mesh = plsc.VectorSubcoreMesh(core_axis_name="c", subcore_axis_name="s",
                                  num_cores=NC, num_subcores=NS)
    f = pl.kernel(
        _tec_body,
        out_type=jax.ShapeDtypeStruct((R * K,), jnp.float32),
        mesh=mesh,
        compiler_params=pltpu.CompilerParams(needs_layout_passes=False),
        scratch_types=[
            pltpu.VMEM((N,), jnp.float32),
            pltpu.VMEM((N,), jnp.float32),
            pltpu.VMEM((ROWS_PER_W * K,), jnp.float32),
            pltpu.VMEM((NGRP * L,), jnp.float32),
            pltpu.SemaphoreType.DMA,
        ],
    )
    return f(x.reshape(R * N)).reshape(R, K)


# speedup vs baseline: 1.2848x; 1.2848x over previous
import jax
import jax.numpy as jnp
from jax import lax
from jax.experimental import pallas as pl
from jax.experimental.pallas import tpu as pltpu
from jax.experimental.pallas import tpu_sc as plsc

R = 128
N = 32768
K = 64
NC = 2
NS = 16
NW = NC * NS
ROWS_PER_W = R // NW
L = 16
NGRP = 16
STRIDE = NGRP * L
SEGLEN = N // STRIDE
NJ = SEGLEN // L
P1_UNROLL = 4

NEG_INF = float("-inf")


def _treemax(vs):
    while len(vs) > 1:
        vs = [jnp.maximum(vs[i], vs[i + 1]) for i in range(0, len(vs) - 1, 2)] + (
            [vs[-1]] if len(vs) % 2 else [])
    return vs[0]


def _tec_body(x_hbm, out_hbm, buf0, buf1, outbuf, smax, sem):
    wid = lax.axis_index("s") * NC + lax.axis_index("c")
    iota = lax.iota(jnp.int32, L)
    neg_vec = jnp.full((L,), NEG_INF, jnp.float32)
    lane0 = iota == 0
    row0 = wid * ROWS_PER_W

    def row_slice(r):
        return x_hbm.at[row0 + r]

    bufs = [buf0, buf1]
    pltpu.sync_copy(row_slice(0), buf0)

    for r in range(ROWS_PER_W):
        rowbuf = bufs[r % 2]
        cp = None
        if r + 1 < ROWS_PER_W:
            cp = pltpu.make_async_copy(row_slice(r + 1), bufs[(r + 1) % 2],
                                       sem)
            cp.start()

        def p1_body(j, ms):
            ms = list(ms)
            for u in range(P1_UNROLL):
                base = pl.multiple_of((j * P1_UNROLL + u) * STRIDE, STRIDE)
                for g in range(NGRP):
                    ms[g] = jnp.maximum(ms[g], rowbuf[pl.ds(base + g * L, L)])
            return tuple(ms)

        init = tuple(jnp.full((L,), NEG_INF, jnp.float32) for _ in range(NGRP))
        segmax = lax.fori_loop(0, SEGLEN // P1_UNROLL, p1_body, init)

        t = jnp.full((L,), NEG_INF, jnp.float32)
        for g in range(NGRP):
            smax[pl.ds(g * L, L)] = segmax[g]
            t = jnp.where(iota == g, jnp.max(segmax[g]), t)

        def ext_body(i, carry):
            t, o0, o1, o2, o3 = carry
            gm = jnp.max(t)
            g_spl = plsc.all_reduce_ffs(t >= gm) + jnp.zeros((L,), jnp.int32)
            gvec = plsc.load_gather(smax, [g_spl * L + iota])
            l_spl = plsc.all_reduce_ffs(gvec >= gm) + jnp.zeros((L,), jnp.int32)
            base = g_spl * L + l_spl

            done = jnp.zeros((L,), jnp.bool_)
            kidx = jnp.zeros((L,), jnp.int32)
            kmask = jnp.zeros((L,), jnp.bool_)
            nv = []
            for ja in range(NJ):
                idx = (ja * L + iota) * STRIDE + base
                v = plsc.load_gather(rowbuf, [idx])
                eq = v >= gm
                hit = plsc.all_reduce_population_count(eq) > 0
                take = hit & (~done)
                f = plsc.all_reduce_ffs(eq)
                km = (iota == f) & take
                done = done | hit
                kidx = jnp.where(km, idx, kidx)
                kmask = kmask | km
                nv.append(jnp.where(km, neg_vec, v))
            plsc.store_scatter(rowbuf, [kidx], neg_vec, mask=kmask)
            newmax_s = jnp.max(_treemax(nv))

            gvec2 = jnp.where(iota == l_spl, newmax_s, gvec)
            plsc.store_scatter(smax, [g_spl * L + iota], gvec2)
            t = jnp.where(iota == g_spl, jnp.max(gvec2), t)

            o0 = jnp.where((i < 16) & (iota == i), gm, o0)
            o1 = jnp.where((i >= 16) & (i < 32) & (iota == i - 16), gm, o1)
            o2 = jnp.where((i >= 32) & (i < 48) & (iota == i - 32), gm, o2)
            o3 = jnp.where((i >= 48) & (iota == i - 48), gm, o3)
            return t, o0, o1, o2, o3

        z = jnp.full((L,), NEG_INF, jnp.float32)
        t, o0, o1, o2, o3 = lax.fori_loop(0, K, ext_body, (t, z, z, z, z))
        outbuf[r, pl.ds(0, L)] = o0
        outbuf[r, pl.ds(16, L)] = o1
        outbuf[r, pl.ds(32, L)] = o2
        outbuf[r, pl.ds(48, L)] = o3

        if cp is not None:
            cp.wait()

    pltpu.sync_copy(
        outbuf, out_hbm.at[pl.ds(row0, ROWS_PER_W)])


def kernel(x):
    mesh = plsc.VectorSubcoreMesh(core_axis_name="c", subcore_axis_name="s",
                                  num_cores=NC, num_subcores=NS)
    f = pl.kernel(
        _tec_body,
        out_type=jax.ShapeDtypeStruct((R, K), jnp.float32),
        mesh=mesh,
        compiler_params=pltpu.CompilerParams(needs_layout_passes=False),
        scratch_types=[
            pltpu.VMEM((N,), jnp.float32),
            pltpu.VMEM((N,), jnp.float32),
            pltpu.VMEM((ROWS_PER_W, K), jnp.float32),
            pltpu.VMEM((NGRP * L,), jnp.float32),
            pltpu.SemaphoreType.DMA,
        ],
    )
    return f(x)
